# baseline (speedup 1.0000x reference)
import jax
import jax.numpy as jnp
from jax.experimental import pallas as pl
from jax.experimental.pallas import tpu as pltpu

_NUM_CLASSES = 81
_B, _A = 32, 8732
_CH = 512
_GRID = (_A + _CH - 1) // _CH


def _ce_pass(x_ref, t_ref, ce_ref):
    x = x_ref[...]
    tgt = t_ref[...]

    m = jnp.max(x, axis=0)
    s = jnp.sum(jnp.exp(x - m[None]), axis=0)
    lse = m + jnp.log(s)

    cls_iota = jax.lax.broadcasted_iota(jnp.int32, x.shape, 0)
    st = jnp.clip(tgt, 0, _NUM_CLASSES - 1)
    tl = jnp.sum(jnp.where(cls_iota == st[None], x, 0.0), axis=0)
    ce = lse - tl
    ce_ref[...] = jnp.where(tgt < 0, 0.0, ce)


def _final_pass(ce_ref, t_ref, lp_ref, lt_ref, out_ref):
    ce = ce_ref[...]
    tgt = t_ref[...]
    pos = tgt > 0

    d = lp_ref[...] - lt_ref[...]
    ad = jnp.abs(d)
    h = jnp.where(ad < 1.0, 0.5 * d * d, ad - 0.5)
    loc_sum = jnp.sum(jnp.where(pos[:, None, :], h, 0.0))

    posf = pos.astype(jnp.float32)
    npos_tot = jnp.sum(posf)
    posce = jnp.sum(jnp.where(pos, ce, 0.0))

    npos_row = jnp.sum(pos.astype(jnp.int32), axis=1, keepdims=True)
    k = 3 * npos_row

    cen = jnp.where(pos, -1.0, ce)
    bits = jax.lax.bitcast_convert_type(cen, jnp.int32)
    t = jnp.zeros((_B, 1), jnp.int32)
    for b in range(30, -1, -1):
        t_try = t | (1 << b)
        cnt = jnp.sum((bits >= t_try).astype(jnp.int32), axis=1, keepdims=True)
        t = jnp.where(cnt >= k, t_try, t)

    gt = bits > t
    c_gt = jnp.sum(gt.astype(jnp.int32), axis=1, keepdims=True)
    sum_gt = jnp.sum(jnp.where(gt, cen, 0.0), axis=1, keepdims=True)
    t_f = jax.lax.bitcast_convert_type(t, jnp.float32)
    rem = (k - c_gt).astype(jnp.float32)
    topk = sum_gt + jnp.where(rem > 0, rem * t_f, 0.0)

    cls_sum = posce + jnp.sum(topk)
    out_ref[...] = ((loc_sum + cls_sum) / npos_tot).reshape(1, 1)


def kernel(loc_preds, loc_targets, cls_preds, cls_targets):
    xt = jnp.transpose(cls_preds, (2, 0, 1))
    lpt = jnp.transpose(loc_preds, (0, 2, 1))
    ltt = jnp.transpose(loc_targets, (0, 2, 1))
    ti = cls_targets.astype(jnp.int32)

    ce = pl.pallas_call(
        _ce_pass,
        grid=(_GRID,),
        in_specs=[
            pl.BlockSpec((_NUM_CLASSES, _B, _CH), lambda i: (0, 0, i)),
            pl.BlockSpec((_B, _CH), lambda i: (0, i)),
        ],
        out_specs=pl.BlockSpec((_B, _CH), lambda i: (0, i)),
        out_shape=jax.ShapeDtypeStruct((_B, _A), jnp.float32),
        compiler_params=pltpu.CompilerParams(
            dimension_semantics=("parallel",),
        ),
    )(xt, ti)

    out = pl.pallas_call(
        _final_pass,
        out_shape=jax.ShapeDtypeStruct((1, 1), jnp.float32),
    )(ce, ti, lpt, ltt)

    return out[0, 0]


# speedup vs baseline: 1.0050x; 1.0050x over previous
import jax
import jax.numpy as jnp
from jax.experimental import pallas as pl
from jax.experimental.pallas import tpu as pltpu

_NUM_CLASSES = 81
_B, _A = 32, 8732
_CH = 1024
_GRID = (_A + _CH - 1) // _CH
_RB = 16
_RGRID = _B // _RB


def _ce_pass(x_ref, t_ref, ce_ref):
    x = x_ref[...]
    tgt = t_ref[...]

    m = jnp.max(x, axis=0)
    s = jnp.sum(jnp.exp(x - m[None]), axis=0)
    lse = m + jnp.log(s)

    cls_iota = jax.lax.broadcasted_iota(jnp.int32, x.shape, 0)
    st = jnp.clip(tgt, 0, _NUM_CLASSES - 1)
    tl = jnp.sum(jnp.where(cls_iota == st[None], x, 0.0), axis=0)
    ce = lse - tl
    ce_ref[...] = jnp.where(tgt < 0, 0.0, ce)


def _select_pass(ce_ref, t_ref, lp_ref, lt_ref,
                 locp_ref, poscep_ref, nposp_ref, topkp_ref):
    ce = ce_ref[...]
    tgt = t_ref[...]
    pos = tgt > 0

    d = lp_ref[...] - lt_ref[...]
    ad = jnp.abs(d)
    h = jnp.where(ad < 1.0, 0.5 * d * d, ad - 0.5)
    loc_sum = jnp.sum(jnp.where(pos[:, None, :], h, 0.0))

    posf = pos.astype(jnp.float32)
    npos_tot = jnp.sum(posf)
    posce = jnp.sum(jnp.where(pos, ce, 0.0))

    npos_row = jnp.sum(pos.astype(jnp.int32), axis=1, keepdims=True)
    k = 3 * npos_row

    cen = jnp.where(pos, -1.0, ce)
    bits = jax.lax.bitcast_convert_type(cen, jnp.int32)
    t = jnp.zeros((_RB, 1), jnp.int32)
    for b in range(30, -1, -1):
        t_try = t | (1 << b)
        cnt = jnp.sum((bits >= t_try).astype(jnp.int32), axis=1, keepdims=True)
        t = jnp.where(cnt >= k, t_try, t)

    gt = bits > t
    c_gt = jnp.sum(gt.astype(jnp.int32), axis=1, keepdims=True)
    sum_gt = jnp.sum(jnp.where(gt, cen, 0.0), axis=1, keepdims=True)
    t_f = jax.lax.bitcast_convert_type(t, jnp.float32)
    rem = (k - c_gt).astype(jnp.float32)
    topk = sum_gt + jnp.where(rem > 0, rem * t_f, 0.0)

    locp_ref[...] = loc_sum.reshape(1, 1, 1)
    poscep_ref[...] = posce.reshape(1, 1, 1)
    nposp_ref[...] = npos_tot.reshape(1, 1, 1)
    topkp_ref[...] = jnp.sum(topk).reshape(1, 1, 1)


def _combine_pass(locp_ref, poscep_ref, nposp_ref, topkp_ref, out_ref):
    loc_sum = jnp.sum(locp_ref[...])
    cls_sum = jnp.sum(poscep_ref[...]) + jnp.sum(topkp_ref[...])
    npos_tot = jnp.sum(nposp_ref[...])
    out_ref[...] = ((loc_sum + cls_sum) / npos_tot).reshape(1, 1)


def kernel(loc_preds, loc_targets, cls_preds, cls_targets):
    xt = jnp.transpose(cls_preds, (2, 0, 1))
    lpt = jnp.transpose(loc_preds, (0, 2, 1))
    ltt = jnp.transpose(loc_targets, (0, 2, 1))
    ti = cls_targets.astype(jnp.int32)

    ce = pl.pallas_call(
        _ce_pass,
        grid=(_GRID,),
        in_specs=[
            pl.BlockSpec((_NUM_CLASSES, _B, _CH), lambda i: (0, 0, i)),
            pl.BlockSpec((_B, _CH), lambda i: (0, i)),
        ],
        out_specs=pl.BlockSpec((_B, _CH), lambda i: (0, i)),
        out_shape=jax.ShapeDtypeStruct((_B, _A), jnp.float32),
        compiler_params=pltpu.CompilerParams(
            dimension_semantics=("parallel",),
        ),
    )(xt, ti)

    part = jax.ShapeDtypeStruct((_RGRID, 1, 1), jnp.float32)
    locp, poscep, nposp, topkp = pl.pallas_call(
        _select_pass,
        grid=(_RGRID,),
        in_specs=[
            pl.BlockSpec((_RB, _A), lambda i: (i, 0)),
            pl.BlockSpec((_RB, _A), lambda i: (i, 0)),
            pl.BlockSpec((_RB, 4, _A), lambda i: (i, 0, 0)),
            pl.BlockSpec((_RB, 4, _A), lambda i: (i, 0, 0)),
        ],
        out_specs=[
            pl.BlockSpec((1, 1, 1), lambda i: (i, 0, 0)),
            pl.BlockSpec((1, 1, 1), lambda i: (i, 0, 0)),
            pl.BlockSpec((1, 1, 1), lambda i: (i, 0, 0)),
            pl.BlockSpec((1, 1, 1), lambda i: (i, 0, 0)),
        ],
        out_shape=[part, part, part, part],
        compiler_params=pltpu.CompilerParams(
            dimension_semantics=("parallel",),
        ),
    )(ce, ti, lpt, ltt)

    out = pl.pallas_call(
        _combine_pass,
        out_shape=jax.ShapeDtypeStruct((1, 1), jnp.float32),
    )(locp, poscep, nposp, topkp)

    return out[0, 0]


# speedup vs baseline: 1.0238x; 1.0187x over previous
import jax
import jax.numpy as jnp
from jax.experimental import pallas as pl
from jax.experimental.pallas import tpu as pltpu

_NUM_CLASSES = 81
_B, _A = 32, 8732
_CH = 1024
_GRID = (_A + _CH - 1) // _CH


def _ce_pass(x_ref, t_ref, ce_ref):
    x = x_ref[...]
    tgt = t_ref[...]

    m = jnp.max(x, axis=0)
    s = jnp.sum(jnp.exp(x - m[None]), axis=0)
    lse = m + jnp.log(s)

    cls_iota = jax.lax.broadcasted_iota(jnp.int32, x.shape, 0)
    st = jnp.clip(tgt, 0, _NUM_CLASSES - 1)
    tl = jnp.sum(jnp.where(cls_iota == st[None], x, 0.0), axis=0)
    ce = lse - tl
    ce_ref[...] = jnp.where(tgt < 0, 0.0, ce)


def _final_pass(ce_ref, t_ref, lp_ref, lt_ref, out_ref):
    ce = ce_ref[...]
    tgt = t_ref[...]
    pos = tgt > 0

    d = lp_ref[...] - lt_ref[...]
    ad = jnp.abs(d)
    h = jnp.where(ad < 1.0, 0.5 * d * d, ad - 0.5)
    loc_sum = jnp.sum(jnp.where(pos[:, None, :], h, 0.0))

    posf = pos.astype(jnp.float32)
    npos_tot = jnp.sum(posf)
    posce = jnp.sum(jnp.where(pos, ce, 0.0))

    npos_row = jnp.sum(pos.astype(jnp.int32), axis=1, keepdims=True)
    k = 3 * npos_row

    cen = jnp.where(pos, -1.0, ce)
    bits = jax.lax.bitcast_convert_type(cen, jnp.int32)

    mb = jnp.maximum(jnp.max(bits), 0)
    bit_hi = jnp.int32(0)
    for j in range(30, -1, -1):
        bit_hi = jnp.where((mb >> j) > 0, jnp.maximum(bit_hi, j), bit_hi)

    def _step(i, t):
        b = bit_hi - i
        t_try = t | (1 << b)
        cnt = jnp.sum((bits >= t_try).astype(jnp.int32), axis=1, keepdims=True)
        return jnp.where(cnt >= k, t_try, t)

    t = jax.lax.fori_loop(0, bit_hi + 1, _step,
                          jnp.zeros((_B, 1), jnp.int32))

    gt = bits > t
    c_gt = jnp.sum(gt.astype(jnp.int32), axis=1, keepdims=True)
    sum_gt = jnp.sum(jnp.where(gt, cen, 0.0), axis=1, keepdims=True)
    t_f = jax.lax.bitcast_convert_type(t, jnp.float32)
    rem = (k - c_gt).astype(jnp.float32)
    topk = sum_gt + jnp.where(rem > 0, rem * t_f, 0.0)

    cls_sum = posce + jnp.sum(topk)
    out_ref[...] = ((loc_sum + cls_sum) / npos_tot).reshape(1, 1)


def kernel(loc_preds, loc_targets, cls_preds, cls_targets):
    xt = jnp.transpose(cls_preds, (2, 0, 1))
    lpt = jnp.transpose(loc_preds, (0, 2, 1))
    ltt = jnp.transpose(loc_targets, (0, 2, 1))
    ti = cls_targets.astype(jnp.int32)

    ce = pl.pallas_call(
        _ce_pass,
        grid=(_GRID,),
        in_specs=[
            pl.BlockSpec((_NUM_CLASSES, _B, _CH), lambda i: (0, 0, i)),
            pl.BlockSpec((_B, _CH), lambda i: (0, i)),
        ],
        out_specs=pl.BlockSpec((_B, _CH), lambda i: (0, i)),
        out_shape=jax.ShapeDtypeStruct((_B, _A), jnp.float32),
        compiler_params=pltpu.CompilerParams(
            dimension_semantics=("parallel",),
        ),
    )(xt, ti)

    out = pl.pallas_call(
        _final_pass,
        out_shape=jax.ShapeDtypeStruct((1, 1), jnp.float32),
    )(ce, ti, lpt, ltt)

    return out[0, 0]


# speedup vs baseline: 1.0815x; 1.0564x over previous
import jax
import jax.numpy as jnp
from jax.experimental import pallas as pl
from jax.experimental.pallas import tpu as pltpu

_NUM_CLASSES = 81
_B, _A = 32, 8732
_CH = 1024
_GRID = (_A + _CH - 1) // _CH


def _ce_pass(x_ref, t_ref, ce_ref):
    x = x_ref[...]
    tgt = t_ref[...]

    m = jnp.max(x, axis=0)
    s = jnp.sum(jnp.exp(x - m[None]), axis=0)
    lse = m + jnp.log(s)

    cls_iota = jax.lax.broadcasted_iota(jnp.int32, x.shape, 0)
    st = jnp.clip(tgt, 0, _NUM_CLASSES - 1)
    tl = jnp.sum(jnp.where(cls_iota == st[None], x, 0.0), axis=0)
    ce = lse - tl
    ce_ref[...] = jnp.where(tgt < 0, 0.0, ce)


def _final_pass(ce_ref, t_ref, lp_ref, lt_ref, out_ref):
    ce = ce_ref[...]
    tgt = t_ref[...]
    pos = tgt > 0

    d = lp_ref[...] - lt_ref[...]
    ad = jnp.abs(d)
    h = jnp.where(ad < 1.0, 0.5 * d * d, ad - 0.5)
    loc_sum = jnp.sum(jnp.where(pos[:, None, :], h, 0.0))

    posf = pos.astype(jnp.float32)
    npos_tot = jnp.sum(posf)
    posce = jnp.sum(jnp.where(pos, ce, 0.0))

    npos_row = jnp.sum(pos.astype(jnp.int32), axis=1, keepdims=True)
    k = 3 * npos_row

    cen = jnp.where(pos, -1.0, ce)
    bits = jax.lax.bitcast_convert_type(cen, jnp.int32)

    t = jnp.zeros((_B, 1), jnp.int32)
    for b in range(30, -1, -1):
        t_try = t | (1 << b)
        cnt = jnp.sum((bits >= t_try).astype(jnp.int32), axis=1, keepdims=True)
        t = jnp.where(cnt >= k, t_try, t)

    gt = bits > t
    c_gt = jnp.sum(gt.astype(jnp.int32), axis=1, keepdims=True)
    sum_gt = jnp.sum(jnp.where(gt, cen, 0.0), axis=1, keepdims=True)
    t_f = jax.lax.bitcast_convert_type(t, jnp.float32)
    rem = (k - c_gt).astype(jnp.float32)
    topk = sum_gt + jnp.where(rem > 0, rem * t_f, 0.0)

    cls_sum = posce + jnp.sum(topk)
    out_ref[...] = ((loc_sum + cls_sum) / npos_tot).reshape(1, 1)


def kernel(loc_preds, loc_targets, cls_preds, cls_targets):
    xt = jnp.transpose(cls_preds, (2, 0, 1))
    lpt = jnp.transpose(loc_preds, (0, 2, 1))
    ltt = jnp.transpose(loc_targets, (0, 2, 1))
    ti = cls_targets.astype(jnp.int32)

    ce = pl.pallas_call(
        _ce_pass,
        grid=(_GRID,),
        in_specs=[
            pl.BlockSpec((_NUM_CLASSES, _B, _CH), lambda i: (0, 0, i)),
            pl.BlockSpec((_B, _CH), lambda i: (0, i)),
        ],
        out_specs=pl.BlockSpec((_B, _CH), lambda i: (0, i)),
        out_shape=jax.ShapeDtypeStruct((_B, _A), jnp.float32),
        compiler_params=pltpu.CompilerParams(
            dimension_semantics=("parallel",),
        ),
    )(xt, ti)

    out = pl.pallas_call(
        _final_pass,
        out_shape=jax.ShapeDtypeStruct((1, 1), jnp.float32),
    )(ce, ti, lpt, ltt)

    return out[0, 0]


# speedup vs baseline: 1.1024x; 1.0194x over previous
import jax
import jax.numpy as jnp
from jax.experimental import pallas as pl
from jax.experimental.pallas import tpu as pltpu

_NUM_CLASSES = 81
_B, _A = 32, 8732
_CH = 1280
_GRID = (_A + _CH - 1) // _CH


def _ce_pass(x_ref, t_ref, ce_ref):
    x = x_ref[...]
    tgt = t_ref[...]

    m = jnp.max(x, axis=0)
    s = jnp.sum(jnp.exp(x - m[None]), axis=0)
    lse = m + jnp.log(s)

    cls_iota = jax.lax.broadcasted_iota(jnp.int32, x.shape, 0)
    st = jnp.clip(tgt, 0, _NUM_CLASSES - 1)
    tl = jnp.sum(jnp.where(cls_iota == st[None], x, 0.0), axis=0)
    ce = lse - tl
    ce_ref[...] = jnp.where(tgt < 0, 0.0, ce)


def _final_pass(ce_ref, t_ref, lp_ref, lt_ref, out_ref):
    ce = ce_ref[...]
    tgt = t_ref[...]
    pos = tgt > 0

    d = lp_ref[...] - lt_ref[...]
    ad = jnp.abs(d)
    h = jnp.where(ad < 1.0, 0.5 * d * d, ad - 0.5)
    loc_sum = jnp.sum(jnp.where(pos[:, None, :], h, 0.0))

    posf = pos.astype(jnp.float32)
    npos_tot = jnp.sum(posf)
    posce = jnp.sum(jnp.where(pos, ce, 0.0))

    npos_row = jnp.sum(pos.astype(jnp.int32), axis=1, keepdims=True)
    k = 3 * npos_row

    cen = jnp.where(pos, -1.0, ce)
    bits = jax.lax.bitcast_convert_type(cen, jnp.int32)

    t = jnp.zeros((_B, 1), jnp.int32)
    for b in range(30, -1, -1):
        t_try = t | (1 << b)
        cnt = jnp.sum((bits >= t_try).astype(jnp.int32), axis=1, keepdims=True)
        t = jnp.where(cnt >= k, t_try, t)

    gt = bits > t
    c_gt = jnp.sum(gt.astype(jnp.int32), axis=1, keepdims=True)
    sum_gt = jnp.sum(jnp.where(gt, cen, 0.0), axis=1, keepdims=True)
    t_f = jax.lax.bitcast_convert_type(t, jnp.float32)
    rem = (k - c_gt).astype(jnp.float32)
    topk = sum_gt + jnp.where(rem > 0, rem * t_f, 0.0)

    cls_sum = posce + jnp.sum(topk)
    out_ref[...] = ((loc_sum + cls_sum) / npos_tot).reshape(1, 1)


def kernel(loc_preds, loc_targets, cls_preds, cls_targets):
    xt = jnp.transpose(cls_preds, (2, 0, 1))
    lpt = jnp.transpose(loc_preds, (0, 2, 1))
    ltt = jnp.transpose(loc_targets, (0, 2, 1))
    ti = cls_targets.astype(jnp.int32)

    ce = pl.pallas_call(
        _ce_pass,
        grid=(_GRID,),
        in_specs=[
            pl.BlockSpec((_NUM_CLASSES, _B, _CH), lambda i: (0, 0, i)),
            pl.BlockSpec((_B, _CH), lambda i: (0, i)),
        ],
        out_specs=pl.BlockSpec((_B, _CH), lambda i: (0, i)),
        out_shape=jax.ShapeDtypeStruct((_B, _A), jnp.float32),
        compiler_params=pltpu.CompilerParams(
            dimension_semantics=("parallel",),
        ),
    )(xt, ti)

    out = pl.pallas_call(
        _final_pass,
        out_shape=jax.ShapeDtypeStruct((1, 1), jnp.float32),
    )(ce, ti, lpt, ltt)

    return out[0, 0]
